# baseline (speedup 1.0000x reference)
import functools

import jax
import jax.numpy as jnp
from jax import lax
from jax.experimental import pallas as pl
from jax.experimental.pallas import tpu as pltpu
from jax.experimental.pallas import tpu_sc as plsc

NUM_VARS = 64
K = 32
KK = K * K
NUM_CATS = 128
B = 256

_SC_NC = 2
_SC_NS = 16
_NW = _SC_NC * _SC_NS
_ROWS = B * NUM_VARS
_RPW = _ROWS // _NW


@functools.partial(
    pl.kernel,
    out_type=jax.ShapeDtypeStruct((_ROWS, 4 * K), jnp.bfloat16),
    mesh=plsc.VectorSubcoreMesh(core_axis_name="c", subcore_axis_name="s"),
    scratch_types=[
        pltpu.VMEM((_RPW,), jnp.int32),
        pltpu.VMEM((_RPW, 4 * K), jnp.bfloat16),
        pltpu.SemaphoreType.DMA,
    ],
    compiler_params=pltpu.CompilerParams(use_tc_tiling_on_sc=False),
)
def _sc_gather(table_hbm, idx_hbm, out_hbm, idx_v, rows_v, sem):
    wid = lax.axis_index("s") * _SC_NC + lax.axis_index("c")
    base = wid * _RPW
    pltpu.sync_copy(idx_hbm.at[pl.ds(base, _RPW)], idx_v)
    pltpu.async_copy(table_hbm.at[idx_v], rows_v, sem).wait()
    pltpu.sync_copy(rows_v, out_hbm.at[pl.ds(base, _RPW)])


def _prep_body(logits_ref, inputs_ref, table_ref, idx_ref):
    for v in range(NUM_VARS):
        lt = jnp.transpose(logits_ref[v]).astype(jnp.bfloat16)
        table_ref[v * NUM_CATS:(v + 1) * NUM_CATS, 0:K] = lt
    vb = lax.broadcasted_iota(jnp.int32, (NUM_VARS, B), 0) * NUM_CATS
    idx_ref[:] = jnp.transpose(inputs_ref[:]) + vb


def _prep(input_logits, inputs):
    return pl.pallas_call(
        _prep_body,
        out_shape=(
            jax.ShapeDtypeStruct((NUM_VARS * NUM_CATS, 4 * K), jnp.bfloat16),
            jax.ShapeDtypeStruct((NUM_VARS, B), jnp.int32),
        ),
    )(input_logits, inputs)


def _tc_body(mars_ref, w0, w1, w2, w3, w4, w5, root_ref, out_ref,
             s0, s1, s2, s3, s4):
    w_refs = [w0, w1, w2, w3, w4, w5]
    out_bufs = [s0, s1, s2, s3, s4, None]
    in_buf = mars_ref
    final = None
    for lvl in range(6):
        rh = NUM_VARS >> (lvl + 1)
        w_ref = w_refs[lvl]
        for r in range(rh):
            if lvl == 0:
                v0, v1 = 2 * r, 2 * r + 1
                left = jnp.transpose(
                    in_buf[v0 * B:(v0 + 1) * B, 0:K].astype(jnp.float32))
                right = jnp.transpose(
                    in_buf[v1 * B:(v1 + 1) * B, 0:K].astype(jnp.float32))
            else:
                left = in_buf[2 * r]
                right = in_buf[2 * r + 1]
            ml = jnp.max(left, axis=0, keepdims=True)
            mr = jnp.max(right, axis=0, keepdims=True)
            a = jnp.exp(left - ml).astype(jnp.bfloat16)
            c = jnp.exp(right - mr).astype(jnp.bfloat16)
            p = (a[:, None, :] * c[None, :, :]).reshape(KK, B)
            ew = jnp.exp(w_ref[r]).astype(jnp.bfloat16)
            o = lax.dot_general(ew, p, (((1,), (0,)), ((), ())),
                                preferred_element_type=jnp.float32)
            res = jnp.log(o + 1e-30) + ml + mr
            if lvl < 5:
                out_bufs[lvl][r] = res
            else:
                final = res
        in_buf = out_bufs[lvl]
    t = final + root_ref[:]
    m = jnp.max(t, axis=0, keepdims=True)
    out_ref[:] = jnp.log(jnp.sum(jnp.exp(t - m), axis=0, keepdims=True)) + m


def _tc_levels(mars0, ws, root_w):
    out = pl.pallas_call(
        _tc_body,
        out_shape=jax.ShapeDtypeStruct((1, B), jnp.float32),
        scratch_shapes=[
            pltpu.VMEM((NUM_VARS >> (l + 1), K, B), jnp.float32)
            for l in range(5)
        ],
    )(mars0, *ws, root_w)
    return out.reshape(B, 1)


def kernel(inputs, input_logits, root_w, sum_w_0, sum_w_1, sum_w_2,
           sum_w_3, sum_w_4, sum_w_5):
    table, idx = _prep(input_logits, inputs.astype(jnp.int32))
    mars0 = _sc_gather(table, idx.reshape(_ROWS))
    ws = [sum_w_0, sum_w_1, sum_w_2, sum_w_3, sum_w_4, sum_w_5]
    return _tc_levels(mars0, ws, root_w.reshape(K, 1))


# speedup vs baseline: 1.6815x; 1.6815x over previous
import functools

import jax
import jax.numpy as jnp
from jax import lax
from jax.experimental import pallas as pl
from jax.experimental.pallas import tpu as pltpu
from jax.experimental.pallas import tpu_sc as plsc

NUM_VARS = 64
K = 32
KK = K * K
NUM_CATS = 128
B = 256

_SC_NC = 2
_SC_NS = 16
_NW = _SC_NC * _SC_NS
_ROWS = B * NUM_VARS
_RPW = _ROWS // _NW


@functools.partial(
    pl.kernel,
    out_type=jax.ShapeDtypeStruct((_ROWS, 4 * K), jnp.float32),
    mesh=plsc.VectorSubcoreMesh(core_axis_name="c", subcore_axis_name="s"),
    scratch_types=[
        pltpu.VMEM((_RPW,), jnp.int32),
        pltpu.VMEM((_RPW, K), jnp.float32),
        pltpu.SemaphoreType.DMA,
    ],
    compiler_params=pltpu.CompilerParams(use_tc_tiling_on_sc=False),
)
def _sc_gather(table_hbm, idx_hbm, out_hbm, idx_v, rows_v, sem):
    wid = lax.axis_index("s") * _SC_NC + lax.axis_index("c")
    base = wid * _RPW
    pltpu.sync_copy(idx_hbm.at[pl.ds(base, _RPW)], idx_v)
    pltpu.async_copy(table_hbm.at[idx_v], rows_v, sem).wait()
    pltpu.sync_copy(rows_v, out_hbm.at[pl.ds(base, _RPW), pl.ds(0, K)])


def _prep_body(logits_ref, inputs_ref, table_ref, idx_ref):
    for v in range(NUM_VARS):
        lt = jnp.transpose(logits_ref[v])
        table_ref[v * NUM_CATS:(v + 1) * NUM_CATS, 0:K] = lt
    vb = lax.broadcasted_iota(jnp.int32, (NUM_VARS, B), 0) * NUM_CATS
    idx_ref[:] = (jnp.transpose(inputs_ref[:]) + vb) * 4


def _prep(input_logits, inputs):
    return pl.pallas_call(
        _prep_body,
        out_shape=(
            jax.ShapeDtypeStruct((NUM_VARS * NUM_CATS, 4 * K), jnp.float32),
            jax.ShapeDtypeStruct((NUM_VARS, B), jnp.int32),
        ),
    )(input_logits, inputs)


def _tc_body(mars_ref, w0, w1, w2, w3, w4, w5, root_ref, out_ref,
             s0, s1, s2, s3, s4):
    w_refs = [w0, w1, w2, w3, w4, w5]
    out_bufs = [s0, s1, s2, s3, s4, None]
    in_buf = mars_ref
    final = None
    for lvl in range(6):
        rh = NUM_VARS >> (lvl + 1)
        w_ref = w_refs[lvl]
        for r in range(rh):
            if lvl == 0:
                v0, v1 = 2 * r, 2 * r + 1
                left = jnp.transpose(in_buf[v0 * B:(v0 + 1) * B, 0:K])
                right = jnp.transpose(in_buf[v1 * B:(v1 + 1) * B, 0:K])
            else:
                left = in_buf[2 * r]
                right = in_buf[2 * r + 1]
            ml = jnp.max(left, axis=0, keepdims=True)
            mr = jnp.max(right, axis=0, keepdims=True)
            a = jnp.exp(left - ml).astype(jnp.bfloat16)
            c = jnp.exp(right - mr).astype(jnp.bfloat16)
            p = (a[:, None, :] * c[None, :, :]).reshape(KK, B)
            ew = jnp.exp(w_ref[r]).astype(jnp.bfloat16)
            o = lax.dot_general(ew, p, (((1,), (0,)), ((), ())),
                                preferred_element_type=jnp.float32)
            res = jnp.log(o + 1e-30) + ml + mr
            if lvl < 5:
                out_bufs[lvl][r] = res
            else:
                final = res
        in_buf = out_bufs[lvl]
    t = final + root_ref[:]
    m = jnp.max(t, axis=0, keepdims=True)
    out_ref[:] = jnp.log(jnp.sum(jnp.exp(t - m), axis=0, keepdims=True)) + m


def _tc_levels(mars0, ws, root_w):
    out = pl.pallas_call(
        _tc_body,
        out_shape=jax.ShapeDtypeStruct((1, B), jnp.float32),
        scratch_shapes=[
            pltpu.VMEM((NUM_VARS >> (l + 1), K, B), jnp.float32)
            for l in range(5)
        ],
    )(mars0, *ws, root_w)
    return out.reshape(B, 1)


def kernel(inputs, input_logits, root_w, sum_w_0, sum_w_1, sum_w_2,
           sum_w_3, sum_w_4, sum_w_5):
    table, idx = _prep(input_logits, inputs.astype(jnp.int32))
    mars0 = _sc_gather(table.reshape(4 * NUM_VARS * NUM_CATS, K),
                       idx.reshape(_ROWS))
    ws = [sum_w_0, sum_w_1, sum_w_2, sum_w_3, sum_w_4, sum_w_5]
    return _tc_levels(mars0, ws, root_w.reshape(K, 1))


# speedup vs baseline: 1.7452x; 1.0379x over previous
import functools

import jax
import jax.numpy as jnp
from jax import lax
from jax.experimental import pallas as pl
from jax.experimental.pallas import tpu as pltpu
from jax.experimental.pallas import tpu_sc as plsc

NUM_VARS = 64
K = 32
KK = K * K
NUM_CATS = 128
B = 256

_SC_NC = 2
_SC_NS = 16
_NW = _SC_NC * _SC_NS
_ROWS = B * NUM_VARS
_RPW = _ROWS // _NW


@functools.partial(
    pl.kernel,
    out_type=jax.ShapeDtypeStruct((_ROWS, 4 * K), jnp.float32),
    mesh=plsc.VectorSubcoreMesh(core_axis_name="c", subcore_axis_name="s"),
    scratch_types=[
        pltpu.VMEM((_RPW,), jnp.int32),
        pltpu.VMEM((_RPW, K), jnp.float32),
        pltpu.SemaphoreType.DMA,
    ],
    compiler_params=pltpu.CompilerParams(use_tc_tiling_on_sc=False),
)
def _sc_gather(table_hbm, idx_hbm, out_hbm, idx_v, rows_v, sem):
    wid = lax.axis_index("s") * _SC_NC + lax.axis_index("c")
    base = wid * _RPW
    pltpu.sync_copy(idx_hbm.at[pl.ds(base, _RPW)], idx_v)
    pltpu.async_copy(table_hbm.at[idx_v], rows_v, sem).wait()
    pltpu.sync_copy(rows_v, out_hbm.at[pl.ds(base, _RPW), pl.ds(0, K)])


def _prep_body(logits_ref, inputs_ref, table_ref, idx_ref):
    for v in range(NUM_VARS):
        lt = jnp.transpose(logits_ref[v])
        table_ref[v * NUM_CATS:(v + 1) * NUM_CATS, 0:K] = lt
    vb = lax.broadcasted_iota(jnp.int32, (NUM_VARS, B), 0) * NUM_CATS
    tv = (jnp.transpose(inputs_ref[:]) + vb) * 4
    idx_ref[:] = jnp.concatenate(
        [tv[:, None, 0:128], tv[:, None, 128:256]], axis=1).reshape(128, 128)


def _prep(input_logits, inputs):
    return pl.pallas_call(
        _prep_body,
        out_shape=(
            jax.ShapeDtypeStruct((NUM_VARS * NUM_CATS, 4 * K), jnp.float32),
            jax.ShapeDtypeStruct((128, 128), jnp.int32),
        ),
    )(input_logits, inputs)


def _tc_body(mars_ref, w0, w1, w2, w3, w4, w5, root_ref, out_ref,
             s0, s1, s2, s3, s4):
    w_refs = [w0, w1, w2, w3, w4, w5]
    out_bufs = [s0, s1, s2, s3, s4, None]
    in_buf = mars_ref
    final = None
    for lvl in range(6):
        rh = NUM_VARS >> (lvl + 1)
        w_ref = w_refs[lvl]
        for r in range(rh):
            if lvl == 0:
                v0, v1 = 2 * r, 2 * r + 1
                left = jnp.transpose(in_buf[v0 * B:(v0 + 1) * B, 0:K])
                right = jnp.transpose(in_buf[v1 * B:(v1 + 1) * B, 0:K])
            else:
                left = in_buf[2 * r]
                right = in_buf[2 * r + 1]
            ml = jnp.max(left, axis=0, keepdims=True)
            mr = jnp.max(right, axis=0, keepdims=True)
            a = jnp.exp(left - ml).astype(jnp.bfloat16)
            c = jnp.exp(right - mr).astype(jnp.bfloat16)
            p = (a[:, None, :] * c[None, :, :]).reshape(KK, B)
            ew = jnp.exp(w_ref[r]).astype(jnp.bfloat16)
            o = lax.dot_general(ew, p, (((1,), (0,)), ((), ())),
                                preferred_element_type=jnp.float32)
            res = jnp.log(o + 1e-30) + ml + mr
            if lvl < 5:
                out_bufs[lvl][r] = res
            else:
                final = res
        in_buf = out_bufs[lvl]
    t = final + root_ref[:]
    m = jnp.max(t, axis=0, keepdims=True)
    out_ref[:] = jnp.log(jnp.sum(jnp.exp(t - m), axis=0, keepdims=True)) + m


def _tc_levels(mars0, ws, root_w):
    out = pl.pallas_call(
        _tc_body,
        out_shape=jax.ShapeDtypeStruct((1, B), jnp.float32),
        scratch_shapes=[
            pltpu.VMEM((NUM_VARS >> (l + 1), K, B), jnp.float32)
            for l in range(5)
        ],
    )(mars0, *ws, root_w)
    return out.reshape(B, 1)


def kernel(inputs, input_logits, root_w, sum_w_0, sum_w_1, sum_w_2,
           sum_w_3, sum_w_4, sum_w_5):
    table, idx = _prep(input_logits, inputs.astype(jnp.int32))
    mars0 = _sc_gather(table.reshape(4 * NUM_VARS * NUM_CATS, K),
                       idx.reshape(_ROWS))
    ws = [sum_w_0, sum_w_1, sum_w_2, sum_w_3, sum_w_4, sum_w_5]
    return _tc_levels(mars0, ws, root_w.reshape(K, 1))
